# baseline (speedup 1.0000x reference)
import functools

import jax
import jax.numpy as jnp
from jax import lax
from jax.experimental import pallas as pl
from jax.experimental.pallas import tpu as pltpu
from jax.experimental.pallas import tpu_sc as plsc

_LANES = 16
_NC = 2
_NS = 16
_NW = _NC * _NS
_IDX_CHUNK = 128


def _make_center_loss(batch, feat):
    b_per_w = batch // _NW
    n_chunks = b_per_w // _IDX_CHUNK
    mesh = plsc.VectorSubcoreMesh(core_axis_name="c", subcore_axis_name="s")

    @functools.partial(
        pl.kernel,
        mesh=mesh,
        compiler_params=pltpu.CompilerParams(
            use_tc_tiling_on_sc=False,
            needs_layout_passes=False,
        ),
        out_type=jax.ShapeDtypeStruct((_NW, _LANES), jnp.float32),
        scratch_types=[
            pltpu.VMEM((n_chunks, _IDX_CHUNK), jnp.int32),
            pltpu.VMEM((b_per_w, feat), jnp.float32),
            pltpu.VMEM((b_per_w, feat), jnp.float32),
            pltpu.VMEM((_LANES,), jnp.float32),
            pltpu.SemaphoreType.DMA,
            pltpu.SemaphoreType.DMA,
        ],
    )
    def center_loss(feat_hbm, lab_hbm, cent_hbm, out_hbm,
                    idx_v, feats_v, rows_v, acc_v, gsem, fsem):
        wid = lax.axis_index("s") * _NC + lax.axis_index("c")
        base = wid * b_per_w

        pltpu.sync_copy(lab_hbm.at[wid], idx_v)
        fcp = pltpu.async_copy(feat_hbm.at[pl.ds(base, b_per_w)], feats_v,
                               fsem)
        gcps = [
            pltpu.async_copy(
                cent_hbm.at[idx_v.at[k]],
                rows_v.at[pl.ds(k * _IDX_CHUNK, _IDX_CHUNK)],
                gsem,
            )
            for k in range(n_chunks)
        ]
        fcp.wait()
        for gcp in gcps:
            gcp.wait()

        n_half = feat // _LANES

        def body(r, accs):
            out = []
            for h in range(n_half):
                f = feats_v[r, pl.ds(h * _LANES, _LANES)]
                c = rows_v[r, pl.ds(h * _LANES, _LANES)]
                d = f - c
                out.append(accs[h] + d * d)
            return tuple(out)

        zero = jnp.zeros((_LANES,), jnp.float32)
        accs = lax.fori_loop(0, b_per_w, body, (zero,) * n_half)
        total = accs[0]
        for h in range(1, n_half):
            total = total + accs[h]
        acc_v[...] = total
        pltpu.sync_copy(acc_v, out_hbm.at[wid])

    return center_loss


@jax.jit
def kernel(features, labels, centers):
    batch, feat = features.shape
    lab = labels.astype(jnp.int32).reshape(_NW, batch // (_NW * _IDX_CHUNK),
                                           _IDX_CHUNK)
    partials = _make_center_loss(batch, feat)(features, lab, centers)
    return jnp.sum(partials) / batch


# speedup vs baseline: 3.0748x; 3.0748x over previous
import functools

import jax
import jax.numpy as jnp
from jax import lax
from jax.experimental import pallas as pl
from jax.experimental.pallas import tpu as pltpu
from jax.experimental.pallas import tpu_sc as plsc

_LANES = 16
_NC = 2
_NS = 16
_NW = _NC * _NS
_CHUNK = 8
_TILE = 128


def _make_center_loss(batch, feat):
    b_per_w = batch // _NW
    n_pairs = b_per_w // (2 * _CHUNK)
    jbs = feat // 8
    mesh = plsc.VectorSubcoreMesh(core_axis_name="c", subcore_axis_name="s")

    @functools.partial(
        pl.kernel,
        mesh=mesh,
        out_type=jax.ShapeDtypeStruct((_NW * _LANES // _TILE, _TILE),
                                      jnp.float32),
        scratch_types=[
            pltpu.VMEM((b_per_w * _LANES,), jnp.int32),
            pltpu.VMEM((b_per_w * feat,), jnp.float32),
            pltpu.VMEM((2, _CHUNK, jbs, 8, _TILE), jnp.float32),
            pltpu.VMEM((_LANES,), jnp.float32),
            pltpu.SemaphoreType.DMA,
            pltpu.SemaphoreType.DMA,
            pltpu.SemaphoreType.DMA,
        ],
    )
    def center_loss(lab_hbm, feat_f_hbm, cent_hbm, out_hbm,
                    lab_v, ff_v, ring_v, acc_v, gsem0, gsem1, fsem):
        wid = lax.axis_index("s") * _NC + lax.axis_index("c")
        base = wid * b_per_w

        pltpu.sync_copy(
            lab_hbm.at[pl.ds(base * _LANES, b_per_w * _LANES)], lab_v)
        fcp = pltpu.async_copy(
            feat_f_hbm.at[pl.ds(base * feat, b_per_w * feat)], ff_v, fsem)
        gsems = (gsem0, gsem1)
        iot = lax.iota(jnp.int32, _LANES)
        iotf = iot.astype(jnp.float32)

        def lab_at(k):
            off = pl.multiple_of(k * _LANES, _LANES)
            return lab_v[pl.ds(off, _LANES)][0]

        def col128(k):
            return pl.multiple_of((lab_at(k) >> 7) * _TILE, _TILE)

        def fire(c, slot):
            def fi(i, carry):
                pltpu.async_copy(
                    cent_hbm.at[:, :, pl.ds(col128(c * _CHUNK + i), _TILE)],
                    ring_v.at[slot, i], gsems[slot])
                return carry

            lax.fori_loop(0, _CHUNK, fi, 0)

        def drain(c, slot):
            def di(i, carry):
                pltpu.make_async_copy(
                    cent_hbm.at[:, :, pl.ds(col128(c * _CHUNK + i), _TILE)],
                    ring_v.at[slot, i], gsems[slot]).wait()
                return carry

            lax.fori_loop(0, _CHUNK, di, 0)

        def compute(c, slot, acc0, acc1):
            def li_body(i, accs):
                a0, a1 = accs
                k = c * _CHUNK + i
                li = lab_at(k) & (_TILE - 1)
                li16 = pl.multiple_of(li & -_LANES, _LANES)
                lif = jnp.full((_LANES,), li & (_LANES - 1),
                               jnp.int32).astype(jnp.float32)
                ohl = jnp.maximum(1.0 - jnp.abs(iotf - lif), 0.0)
                k32 = pl.multiple_of(k * feat, _LANES)
                flo = ff_v[pl.ds(k32, _LANES)]
                fhi = ff_v[pl.ds(k32 + _LANES, _LANES)]

                def jbody(jb, carry):
                    sc2, w = carry
                    fsel = jnp.where(jb < 2, 1.0, 0.0)
                    vf = flo * fsel + fhi * (1.0 - fsel)
                    odd = jnp.where((jb & 1) == 1, 1.0, 0.0)
                    for jj in range(8):
                        vc = ring_v[slot, i, jb, jj, pl.ds(li16, _LANES)]
                        u = vc * ohl
                        sc2 = sc2 + u * u
                        fj = vf[jj] * (1.0 - odd) + vf[8 + jj] * odd
                        w = w + fj * vc
                    return sc2, w

                z = jnp.zeros((_LANES,), jnp.float32)
                sc2, w = lax.fori_loop(0, jbs, jbody, (z, z))
                a0 = a0 + flo * flo + fhi * fhi + sc2
                a1 = a1 - 2.0 * (w * ohl)
                return a0, a1

            return lax.fori_loop(0, _CHUNK, li_body, (acc0, acc1))

        def body(p, accs):
            acc0, acc1 = accs
            c0 = p * 2
            drain(c0, 0)
            acc0, acc1 = compute(c0, 0, acc0, acc1)

            @pl.when(p + 1 < n_pairs)
            def _():
                fire(c0 + 2, 0)

            drain(c0 + 1, 1)
            acc0, acc1 = compute(c0 + 1, 1, acc0, acc1)

            @pl.when(p + 1 < n_pairs)
            def _():
                fire(c0 + 3, 1)

            return acc0, acc1

        fire(0, 0)
        fire(1, 1)
        fcp.wait()
        zero = jnp.zeros((_LANES,), jnp.float32)
        acc0, acc1 = lax.fori_loop(0, n_pairs, body, (zero, zero))
        acc_v[...] = acc0 + acc1
        pltpu.sync_copy(
            acc_v,
            out_hbm.at[wid // 8, pl.ds((wid % 8) * _LANES, _LANES)])

    return center_loss


@jax.jit
def kernel(features, labels, centers):
    batch, feat = features.shape
    cent3 = centers.T.reshape(feat // 8, 8, centers.shape[0])
    lab_rep = jnp.repeat(labels.astype(jnp.int32), _LANES)
    partials = _make_center_loss(batch, feat)(
        lab_rep, features.reshape(-1), cent3)
    return jnp.sum(partials) / batch


# speedup vs baseline: 3.1188x; 1.0143x over previous
import functools

import jax
import jax.numpy as jnp
from jax import lax
from jax.experimental import pallas as pl
from jax.experimental.pallas import tpu as pltpu
from jax.experimental.pallas import tpu_sc as plsc

_LANES = 16
_NC = 2
_NS = 16
_NW = _NC * _NS
_CHUNK = 8
_TILE = 128


def _make_center_loss(batch, feat):
    b_per_w = batch // _NW
    n_pairs = b_per_w // (2 * _CHUNK)
    jbs = feat // 8
    mesh = plsc.VectorSubcoreMesh(core_axis_name="c", subcore_axis_name="s")

    @functools.partial(
        pl.kernel,
        mesh=mesh,
        out_type=jax.ShapeDtypeStruct((_NW * _LANES // _TILE, _TILE),
                                      jnp.float32),
        scratch_types=[
            pltpu.VMEM((b_per_w * _LANES,), jnp.int32),
            pltpu.VMEM((b_per_w * feat,), jnp.float32),
            pltpu.VMEM((2, _CHUNK, jbs, 8, _TILE), jnp.float32),
            pltpu.VMEM((_LANES,), jnp.float32),
            pltpu.SemaphoreType.DMA,
            pltpu.SemaphoreType.DMA,
            pltpu.SemaphoreType.DMA,
        ],
    )
    def center_loss(lab_hbm, feat_f_hbm, cent_hbm, out_hbm,
                    lab_v, ff_v, ring_v, acc_v, gsem0, gsem1, fsem):
        wid = lax.axis_index("s") * _NC + lax.axis_index("c")
        base = wid * b_per_w

        pltpu.sync_copy(
            lab_hbm.at[pl.ds(base * _LANES, b_per_w * _LANES)], lab_v)
        fcp = pltpu.async_copy(
            feat_f_hbm.at[pl.ds(base * feat, b_per_w * feat)], ff_v, fsem)
        gsems = (gsem0, gsem1)
        iot = lax.iota(jnp.int32, _LANES)
        iotf = iot.astype(jnp.float32)

        def lab_at(k):
            off = pl.multiple_of(k * _LANES, _LANES)
            return lab_v[pl.ds(off, _LANES)][0]

        def col128(k):
            return pl.multiple_of((lab_at(k) >> 7) * _TILE, _TILE)

        def fire(c, slot):
            def fi(i, carry):
                pltpu.async_copy(
                    cent_hbm.at[:, :, pl.ds(col128(c * _CHUNK + i), _TILE)],
                    ring_v.at[slot, i], gsems[slot])
                return carry

            lax.fori_loop(0, _CHUNK, fi, 0)

        def drain(c, slot):
            def di(i, carry):
                pltpu.make_async_copy(
                    cent_hbm.at[:, :, pl.ds(col128(c * _CHUNK + i), _TILE)],
                    ring_v.at[slot, i], gsems[slot]).wait()
                return carry

            lax.fori_loop(0, _CHUNK, di, 0)

        def compute(c, slot, acc0, acc1):
            def li_body(i, accs):
                a0, a1 = accs
                k = c * _CHUNK + i
                li = lab_at(k) & (_TILE - 1)
                li16 = pl.multiple_of(li & -_LANES, _LANES)
                lif = jnp.full((_LANES,), li & (_LANES - 1),
                               jnp.int32).astype(jnp.float32)
                ohl = jnp.maximum(1.0 - jnp.abs(iotf - lif), 0.0)
                k32 = pl.multiple_of(k * feat, _LANES)
                flo = ff_v[pl.ds(k32, _LANES)]
                fhi = ff_v[pl.ds(k32 + _LANES, _LANES)]

                z = jnp.zeros((_LANES,), jnp.float32)
                sc2, w = z, z
                for jb in range(jbs):
                    vf = flo if jb < jbs // 2 else fhi
                    for jj in range(8):
                        vc = ring_v[slot, i, jb, jj, pl.ds(li16, _LANES)]
                        u = vc * ohl
                        sc2 = sc2 + u * u
                        w = w + vf[(jb % 2) * 8 + jj] * vc
                a0 = a0 + flo * flo + fhi * fhi + sc2
                a1 = a1 - 2.0 * (w * ohl)
                return a0, a1

            return lax.fori_loop(0, _CHUNK, li_body, (acc0, acc1))

        def body(p, accs):
            acc0, acc1 = accs
            c0 = p * 2
            drain(c0, 0)
            acc0, acc1 = compute(c0, 0, acc0, acc1)

            @pl.when(p + 1 < n_pairs)
            def _():
                fire(c0 + 2, 0)

            drain(c0 + 1, 1)
            acc0, acc1 = compute(c0 + 1, 1, acc0, acc1)

            @pl.when(p + 1 < n_pairs)
            def _():
                fire(c0 + 3, 1)

            return acc0, acc1

        fire(0, 0)
        fire(1, 1)
        fcp.wait()
        zero = jnp.zeros((_LANES,), jnp.float32)
        acc0, acc1 = lax.fori_loop(0, n_pairs, body, (zero, zero))
        acc_v[...] = acc0 + acc1
        pltpu.sync_copy(
            acc_v,
            out_hbm.at[wid // 8, pl.ds((wid % 8) * _LANES, _LANES)])

    return center_loss


@jax.jit
def kernel(features, labels, centers):
    batch, feat = features.shape
    cent3 = centers.T.reshape(feat // 8, 8, centers.shape[0])
    lab_rep = jnp.repeat(labels.astype(jnp.int32), _LANES)
    partials = _make_center_loss(batch, feat)(
        lab_rep, features.reshape(-1), cent3)
    return jnp.sum(partials) / batch
